# baseline (speedup 1.0000x reference)
import functools

import jax
import jax.numpy as jnp
from jax import lax
from jax.experimental import pallas as pl
from jax.experimental.pallas import tpu as pltpu
from jax.experimental.pallas import tpu_sc as plsc

_MAX_LEN = 2048


def kernel(q, k, emb):
    s_q = q.shape[2]
    s_k = k.shape[2]
    d = emb.shape[1]

    emb_pad = jnp.concatenate([emb, emb[-1:]], axis=0)

    info = plsc.get_sparse_core_info()
    nw = info.num_cores * info.num_subcores

    chunk = 64
    nch = s_k // chunk

    mesh = plsc.VectorSubcoreMesh(core_axis_name="c", subcore_axis_name="s")

    @functools.partial(
        pl.kernel,
        mesh=mesh,
        out_type=jax.ShapeDtypeStruct((s_q * s_k * d,), jnp.float32),
    )
    def run(emb_hbm, out_hbm):
        w = lax.axis_index("s") * info.num_cores + lax.axis_index("c")
        src0 = (_MAX_LEN - w) * d
        dst0 = w * s_k * d
        csz = chunk * d

        def body(ci, carry):
            off = ci * csz
            pltpu.sync_copy(
                emb_hbm.at[pl.ds(src0 + off, csz)],
                out_hbm.at[pl.ds(dst0 + off, csz)],
            )
            return carry

        lax.fori_loop(0, nch, body, 0)

    out = run(emb_pad.reshape(-1))
    return out.reshape(s_q, s_k, d)
